# baseline (speedup 1.0000x reference)
import functools
import math

import jax
import jax.numpy as jnp
from jax import lax
from jax.experimental import pallas as pl
from jax.experimental.pallas import tpu as pltpu
from jax.experimental.pallas import tpu_sc as plsc

N_NODES = 10000
N_EDGES = 320000
HIDDEN = 128
ALPHA = 0.1
THETA = 0.5

NC = 2
NS = 16
NW = NC * NS
CHUNK = 128
NCHUNKS = 80
E_PAD = NW * NCHUNKS * CHUNK
N_PAD = 10240
ROWS_PER_SUB = N_PAD // NS
PAD_DST = N_NODES


def _linrelu_body(x_ref, w_ref, b_ref, o_ref):
    o_ref[...] = jnp.maximum(
        jnp.dot(x_ref[...], w_ref[...], preferred_element_type=jnp.float32)
        + b_ref[...],
        0.0,
    )


def _combine_body(p_ref, x0_ref, w_ref, o_ref, *, beta):
    xx = (1.0 - ALPHA) * (p_ref[0] + p_ref[1]) + ALPHA * x0_ref[...]
    out = (1.0 - beta) * xx + beta * jnp.dot(
        xx, w_ref[...], preferred_element_type=jnp.float32
    )
    o_ref[...] = jnp.maximum(out, 0.0)


_TC_ROWS = 1000


def _tc_linrelu(x, w, b):
    grid = (N_NODES // _TC_ROWS,)
    return pl.pallas_call(
        _linrelu_body,
        grid=grid,
        in_specs=[
            pl.BlockSpec((_TC_ROWS, HIDDEN), lambda i: (i, 0)),
            pl.BlockSpec((HIDDEN, HIDDEN), lambda i: (0, 0)),
            pl.BlockSpec((1, HIDDEN), lambda i: (0, 0)),
        ],
        out_specs=pl.BlockSpec((_TC_ROWS, HIDDEN), lambda i: (i, 0)),
        out_shape=jax.ShapeDtypeStruct((N_NODES, HIDDEN), jnp.float32),
    )(x, w, b)


def _tc_combine(partials, x0, w, beta):
    grid = (N_NODES // _TC_ROWS,)
    return pl.pallas_call(
        functools.partial(_combine_body, beta=beta),
        grid=grid,
        in_specs=[
            pl.BlockSpec((2, _TC_ROWS, HIDDEN), lambda i: (0, i, 0)),
            pl.BlockSpec((_TC_ROWS, HIDDEN), lambda i: (i, 0)),
            pl.BlockSpec((HIDDEN, HIDDEN), lambda i: (0, 0)),
        ],
        out_specs=pl.BlockSpec((_TC_ROWS, HIDDEN), lambda i: (i, 0)),
        out_shape=jax.ShapeDtypeStruct((N_NODES, HIDDEN), jnp.float32),
    )(partials, x0, w)


_SC_MESH = plsc.VectorSubcoreMesh(core_axis_name="c", subcore_axis_name="s")

NBUF = 1
_NGROUP = NCHUNKS // NBUF


@functools.partial(
    pl.kernel,
    out_type=jax.ShapeDtypeStruct((NC, N_PAD, HIDDEN), jnp.float32),
    mesh=_SC_MESH,
    scratch_types=[
        pltpu.VMEM((NCHUNKS, CHUNK), jnp.int32),
        pltpu.VMEM((NCHUNKS, CHUNK), jnp.int32),
        [pltpu.VMEM((CHUNK, HIDDEN), jnp.float32) for _ in range(NBUF)],
        pltpu.VMEM_SHARED((N_PAD, HIDDEN), jnp.float32),
        [pltpu.SemaphoreType.DMA for _ in range(NBUF)],
    ],
)
def _sc_gather_segsum(x_hbm, src_hbm, dst_hbm, zeros_hbm, out_hbm,
                      src_v, dst_v, rows, acc_sh, gsems):
    cid = lax.axis_index("c")
    sid = lax.axis_index("s")
    wid = sid * NC + cid

    pltpu.sync_copy(src_hbm.at[wid], src_v)
    pltpu.sync_copy(dst_hbm.at[wid], dst_v)

    row0 = sid * ROWS_PER_SUB
    pltpu.sync_copy(zeros_hbm, acc_sh.at[pl.ds(row0, ROWS_PER_SUB)])
    plsc.subcore_barrier()

    def body(g, carry):
        descs = [
            pltpu.async_copy(x_hbm.at[src_v.at[g * NBUF + b]], rows[b],
                             gsems[b])
            for b in range(NBUF)
        ]
        for b in range(NBUF):
            descs[b].wait()
            pltpu.sync_copy(rows[b], acc_sh.at[dst_v.at[g * NBUF + b]],
                            add=True)
        return carry

    lax.fori_loop(0, _NGROUP, body, 0)

    plsc.subcore_barrier()
    pltpu.sync_copy(acc_sh.at[pl.ds(row0, ROWS_PER_SUB)],
                    out_hbm.at[cid, pl.ds(row0, ROWS_PER_SUB)])


def kernel(x, edge_index, W_lin, b_lin, W1_1, W1_2):
    ei = edge_index.astype(jnp.int32)
    npad = E_PAD - N_EDGES
    src = jnp.concatenate(
        [ei[0], jnp.zeros((npad,), jnp.int32)]).reshape(NW, NCHUNKS, CHUNK)
    dst = jnp.concatenate(
        [ei[1], jnp.full((npad,), PAD_DST, jnp.int32)]).reshape(NW, NCHUNKS, CHUNK)
    b2 = b_lin.reshape(1, HIDDEN).astype(jnp.float32)
    zeros = jnp.zeros((ROWS_PER_SUB, HIDDEN), jnp.float32)

    h = _tc_linrelu(x, W_lin, b2)
    x_cur = h
    for layer, Wc in enumerate((W1_1, W1_2)):
        beta = math.log(THETA / (layer + 1) + 1.0)
        partials = _sc_gather_segsum(x_cur, src, dst, zeros)
        x_cur = _tc_combine(partials, h, Wc, beta)
    return x_cur


# speedup vs baseline: 1.6379x; 1.6379x over previous
import functools
import math

import jax
import jax.numpy as jnp
from jax import lax
from jax.experimental import pallas as pl
from jax.experimental.pallas import tpu as pltpu
from jax.experimental.pallas import tpu_sc as plsc

N_NODES = 10000
N_EDGES = 320000
HIDDEN = 128
ALPHA = 0.1
THETA = 0.5

NC = 2
NS = 16
NW = NC * NS
CHUNK = 80
NCHUNKS = 126
E_PAD = NW * NCHUNKS * CHUNK
N_PAD = 10240
ROWS_PER_SUB = N_PAD // NS
PAD_DST = N_NODES


def _linrelu_body(x_ref, w_ref, b_ref, o_ref):
    o_ref[...] = jnp.maximum(
        jnp.dot(x_ref[...], w_ref[...], preferred_element_type=jnp.float32)
        + b_ref[...],
        0.0,
    )


def _combine_body(p_ref, x0_ref, w_ref, o_ref, *, beta):
    xx = (1.0 - ALPHA) * (p_ref[0] + p_ref[1]) + ALPHA * x0_ref[...]
    out = (1.0 - beta) * xx + beta * jnp.dot(
        xx, w_ref[...], preferred_element_type=jnp.float32
    )
    o_ref[...] = jnp.maximum(out, 0.0)


_TC_ROWS = 1000


def _tc_linrelu(x, w, b):
    grid = (N_NODES // _TC_ROWS,)
    return pl.pallas_call(
        _linrelu_body,
        grid=grid,
        in_specs=[
            pl.BlockSpec((_TC_ROWS, HIDDEN), lambda i: (i, 0)),
            pl.BlockSpec((HIDDEN, HIDDEN), lambda i: (0, 0)),
            pl.BlockSpec((1, HIDDEN), lambda i: (0, 0)),
        ],
        out_specs=pl.BlockSpec((_TC_ROWS, HIDDEN), lambda i: (i, 0)),
        out_shape=jax.ShapeDtypeStruct((N_NODES, HIDDEN), jnp.float32),
    )(x, w, b)


def _tc_combine(partials, x0, w, beta):
    grid = (N_NODES // _TC_ROWS,)
    return pl.pallas_call(
        functools.partial(_combine_body, beta=beta),
        grid=grid,
        in_specs=[
            pl.BlockSpec((2, _TC_ROWS, HIDDEN), lambda i: (0, i, 0)),
            pl.BlockSpec((_TC_ROWS, HIDDEN), lambda i: (i, 0)),
            pl.BlockSpec((HIDDEN, HIDDEN), lambda i: (0, 0)),
        ],
        out_specs=pl.BlockSpec((_TC_ROWS, HIDDEN), lambda i: (i, 0)),
        out_shape=jax.ShapeDtypeStruct((N_NODES, HIDDEN), jnp.float32),
    )(partials, x0, w)


_SC_MESH = plsc.VectorSubcoreMesh(core_axis_name="c", subcore_axis_name="s")

NBUF = 2
_NGROUP = NCHUNKS // NBUF


@functools.partial(
    pl.kernel,
    out_type=jax.ShapeDtypeStruct((NC, N_PAD, HIDDEN), jnp.float32),
    mesh=_SC_MESH,
    scratch_types=[
        [pltpu.VMEM((CHUNK,), jnp.int32) for _ in range(NBUF)],
        [pltpu.VMEM((CHUNK,), jnp.int32) for _ in range(NBUF)],
        [pltpu.VMEM((CHUNK, HIDDEN), jnp.float32) for _ in range(NBUF)],
        pltpu.VMEM_SHARED((N_PAD, HIDDEN), jnp.float32),
        [pltpu.SemaphoreType.DMA for _ in range(NBUF)],
        [pltpu.SemaphoreType.DMA for _ in range(NBUF)],
    ],
)
def _sc_gather_segsum(x_hbm, src_hbm, dst_hbm, zeros_hbm, out_hbm,
                      srcs, dsts, rows, acc_sh, isems, gsems):
    cid = lax.axis_index("c")
    sid = lax.axis_index("s")
    wid = sid * NC + cid
    base = wid * (NCHUNKS * CHUNK)

    row0 = sid * ROWS_PER_SUB
    pltpu.sync_copy(zeros_hbm, acc_sh.at[pl.ds(row0, ROWS_PER_SUB)])
    plsc.subcore_barrier()

    def body(g, carry):
        idescs = []
        for b in range(NBUF):
            eb = base + (g * NBUF + b) * CHUNK
            idescs.append(
                pltpu.async_copy(src_hbm.at[pl.ds(eb, CHUNK)], srcs[b],
                                 isems[b]))
            idescs.append(
                pltpu.async_copy(dst_hbm.at[pl.ds(eb, CHUNK)], dsts[b],
                                 isems[b]))
        gdescs = []
        for b in range(NBUF):
            idescs[2 * b].wait()
            idescs[2 * b + 1].wait()
            gdescs.append(
                pltpu.async_copy(x_hbm.at[srcs[b]], rows[b], gsems[b]))
        for b in range(NBUF):
            gdescs[b].wait()
            pltpu.sync_copy(rows[b], acc_sh.at[dsts[b]], add=True)
        return carry

    lax.fori_loop(0, _NGROUP, body, 0)

    plsc.subcore_barrier()
    pltpu.sync_copy(acc_sh.at[pl.ds(row0, ROWS_PER_SUB)],
                    out_hbm.at[cid, pl.ds(row0, ROWS_PER_SUB)])


def kernel(x, edge_index, W_lin, b_lin, W1_1, W1_2):
    ei = edge_index.astype(jnp.int32)
    npad = E_PAD - N_EDGES
    src = jnp.concatenate([ei[0], jnp.zeros((npad,), jnp.int32)])
    dst = jnp.concatenate([ei[1], jnp.full((npad,), PAD_DST, jnp.int32)])
    b2 = b_lin.reshape(1, HIDDEN).astype(jnp.float32)
    zeros = jnp.zeros((ROWS_PER_SUB, HIDDEN), jnp.float32)

    h = _tc_linrelu(x, W_lin, b2)
    x_cur = h
    for layer, Wc in enumerate((W1_1, W1_2)):
        beta = math.log(THETA / (layer + 1) + 1.0)
        partials = _sc_gather_segsum(x_cur, src, dst, zeros)
        x_cur = _tc_combine(partials, h, Wc, beta)
    return x_cur


# speedup vs baseline: 2.2163x; 1.3531x over previous
import functools
import math

import jax
import jax.numpy as jnp
from jax import lax
from jax.experimental import pallas as pl
from jax.experimental.pallas import tpu as pltpu
from jax.experimental.pallas import tpu_sc as plsc

N_NODES = 10000
N_EDGES = 320000
HIDDEN = 128
ALPHA = 0.1
THETA = 0.5

NC = 2
NS = 16
NW = NC * NS
EDGES_PER_W = N_EDGES // NW
CHUNK = 80
NCHUNKS = EDGES_PER_W // CHUNK
N_PAD = 10240
ROWS_PER_SUB = N_PAD // NS


def _linrelu_body(x_ref, w_ref, b_ref, o_ref):
    o_ref[...] = jnp.maximum(
        jnp.dot(x_ref[...], w_ref[...], preferred_element_type=jnp.float32)
        + b_ref[...],
        0.0,
    )


def _combine_body(p_ref, x0_ref, w_ref, o_ref, *, beta):
    xx = (1.0 - ALPHA) * (p_ref[0] + p_ref[1]) + ALPHA * x0_ref[...]
    out = (1.0 - beta) * xx + beta * jnp.dot(
        xx, w_ref[...], preferred_element_type=jnp.float32
    )
    o_ref[...] = jnp.maximum(out, 0.0)


_TC_ROWS = 1000


def _tc_linrelu(x, w, b):
    grid = (N_NODES // _TC_ROWS,)
    return pl.pallas_call(
        _linrelu_body,
        grid=grid,
        in_specs=[
            pl.BlockSpec((_TC_ROWS, HIDDEN), lambda i: (i, 0)),
            pl.BlockSpec((HIDDEN, HIDDEN), lambda i: (0, 0)),
            pl.BlockSpec((1, HIDDEN), lambda i: (0, 0)),
        ],
        out_specs=pl.BlockSpec((_TC_ROWS, HIDDEN), lambda i: (i, 0)),
        out_shape=jax.ShapeDtypeStruct((N_NODES, HIDDEN), jnp.float32),
    )(x, w, b)


def _tc_combine(partials, x0, w, beta):
    grid = (N_NODES // _TC_ROWS,)
    return pl.pallas_call(
        functools.partial(_combine_body, beta=beta),
        grid=grid,
        in_specs=[
            pl.BlockSpec((2, _TC_ROWS, HIDDEN), lambda i: (0, i, 0)),
            pl.BlockSpec((_TC_ROWS, HIDDEN), lambda i: (i, 0)),
            pl.BlockSpec((HIDDEN, HIDDEN), lambda i: (0, 0)),
        ],
        out_specs=pl.BlockSpec((_TC_ROWS, HIDDEN), lambda i: (i, 0)),
        out_shape=jax.ShapeDtypeStruct((N_NODES, HIDDEN), jnp.float32),
    )(partials, x0, w)


_SC_MESH = plsc.VectorSubcoreMesh(core_axis_name="c", subcore_axis_name="s")

NBUF = 1
_NGROUP = NCHUNKS // NBUF


@functools.partial(
    pl.kernel,
    out_type=jax.ShapeDtypeStruct((NC, N_PAD, HIDDEN), jnp.float32),
    mesh=_SC_MESH,
    scratch_types=[
        pltpu.VMEM((NCHUNKS, CHUNK), jnp.int32),
        pltpu.VMEM((NCHUNKS, CHUNK), jnp.int32),
        [pltpu.VMEM((CHUNK, HIDDEN), jnp.float32) for _ in range(NBUF)],
        pltpu.VMEM_SHARED((N_PAD, HIDDEN), jnp.float32),
        [pltpu.SemaphoreType.DMA for _ in range(NBUF)],
    ],
)
def _sc_gather_segsum(x_hbm, src_hbm, dst_hbm, zeros_hbm, out_hbm,
                      src_v, dst_v, rows, acc_sh, gsems):
    cid = lax.axis_index("c")
    sid = lax.axis_index("s")
    wid = sid * NC + cid

    pltpu.sync_copy(src_hbm.at[wid], src_v)
    pltpu.sync_copy(dst_hbm.at[wid], dst_v)

    row0 = sid * ROWS_PER_SUB
    pltpu.sync_copy(zeros_hbm, acc_sh.at[pl.ds(row0, ROWS_PER_SUB)])
    plsc.subcore_barrier()

    def body(g, carry):
        descs = [
            pltpu.async_copy(x_hbm.at[src_v.at[g * NBUF + b]], rows[b],
                             gsems[b])
            for b in range(NBUF)
        ]
        for b in range(NBUF):
            descs[b].wait()
            pltpu.sync_copy(rows[b], acc_sh.at[dst_v.at[g * NBUF + b]],
                            add=True)
        return carry

    lax.fori_loop(0, _NGROUP, body, 0)

    plsc.subcore_barrier()
    pltpu.sync_copy(acc_sh.at[pl.ds(row0, ROWS_PER_SUB)],
                    out_hbm.at[cid, pl.ds(row0, ROWS_PER_SUB)])


def kernel(x, edge_index, W_lin, b_lin, W1_1, W1_2):
    ei = edge_index.astype(jnp.int32)
    src = ei[0].reshape(NW, NCHUNKS, CHUNK)
    dst = ei[1].reshape(NW, NCHUNKS, CHUNK)
    b2 = b_lin.reshape(1, HIDDEN).astype(jnp.float32)
    zeros = jnp.zeros((ROWS_PER_SUB, HIDDEN), jnp.float32)

    h = _tc_linrelu(x, W_lin, b2)
    x_cur = h
    for layer, Wc in enumerate((W1_1, W1_2)):
        beta = math.log(THETA / (layer + 1) + 1.0)
        partials = _sc_gather_segsum(x_cur, src, dst, zeros)
        x_cur = _tc_combine(partials, h, Wc, beta)
    return x_cur
